# Initial kernel scaffold; baseline (speedup 1.0000x reference)
import jax
import jax.numpy as jnp
from jax.experimental import pallas as pl


def kernel(x, edge_index, edge_attr, attention, Wz, bz, Lwz, lbz, Wr, br, Lwr, lbr, Wh, bh, Lwh, lbh, Wlin, blin):
    raise NotImplementedError("write your pallas kernel here")


# speedup vs baseline: 2.1555x; 2.1555x over previous
import functools

import jax
import jax.numpy as jnp
from jax.experimental import pallas as pl
from jax.experimental.pallas import tpu as pltpu

_TN = 512


def _dense_body(axp_ref, xp_ref, d2_ref, wzh_ref, bzh_ref, probs_ref,
                wlin_ref, blin_ref, out_ref):
    P = axp_ref.shape[0]
    H = wzh_ref.shape[0]
    d2 = d2_ref[...]
    wzh = wzh_ref[...]
    bzh = bzh_ref[0:1, :]
    acc = jnp.zeros((axp_ref.shape[1], H), jnp.float32)
    for t in range(P):
        y = axp_ref[t] + d2 * xp_ref[t]
        g = jnp.dot(y, wzh, preferred_element_type=jnp.float32) + bzh
        z = jax.nn.sigmoid(g[:, :H])
        ht = jnp.tanh(g[:, H:])
        acc = acc + probs_ref[0, t] * ((1.0 - z) * ht)
    h = jnp.maximum(acc, 0.0)
    out_ref[...] = (jnp.dot(h, wlin_ref[...], preferred_element_type=jnp.float32)
                    + blin_ref[0:1, :])


def _dense_stage(axp, xp, d2, wzh, bzh, probs_pad, wlin_pad, blin_pad):
    P, Npad, H = axp.shape
    grid = (Npad // _TN,)
    return pl.pallas_call(
        _dense_body,
        grid=grid,
        in_specs=[
            pl.BlockSpec((P, _TN, H), lambda i: (0, i, 0)),
            pl.BlockSpec((P, _TN, H), lambda i: (0, i, 0)),
            pl.BlockSpec((_TN, 1), lambda i: (i, 0)),
            pl.BlockSpec((H, 2 * H), lambda i: (0, 0)),
            pl.BlockSpec((8, 2 * H), lambda i: (0, 0)),
            pl.BlockSpec((8, 128), lambda i: (0, 0)),
            pl.BlockSpec((H, 128), lambda i: (0, 0)),
            pl.BlockSpec((8, 128), lambda i: (0, 0)),
        ],
        out_specs=pl.BlockSpec((_TN, 128), lambda i: (i, 0)),
        out_shape=jax.ShapeDtypeStruct((Npad, 128), jnp.float32),
        compiler_params=pltpu.CompilerParams(
            dimension_semantics=("parallel",)),
    )(axp, xp, d2, wzh, bzh, probs_pad, wlin_pad, blin_pad)


def kernel(x, edge_index, edge_attr, attention, Wz, bz, Lwz, lbz, Wr, br,
           Lwr, lbr, Wh, bh, Lwh, lbh, Wlin, blin):
    N, DIN, P = x.shape
    H = Wz.shape[1]
    PO = Wlin.shape[1]
    Npad = ((N + _TN - 1) // _TN) * _TN

    src = edge_index[0]
    dst = edge_index[1]
    w = edge_attr

    deg = jnp.zeros((N,), jnp.float32).at[dst].add(w) + 1.0
    dinv = jax.lax.rsqrt(deg)
    norm = dinv[src] * w * dinv[dst]
    d2 = (dinv * dinv)[:, None]
    d2 = jnp.pad(d2, ((0, Npad - N), (0, 0)))

    wzh = jnp.concatenate([Wz @ Lwz[:H], Wh @ Lwh[:H]], axis=1)
    bzh = jnp.concatenate([bz @ Lwz[:H] + lbz, bh @ Lwh[:H] + lbh])
    bzh = jnp.zeros((8, 2 * H), jnp.float32).at[0].set(bzh)
    probs = jax.nn.softmax(attention)
    probs_pad = jnp.zeros((8, 128), jnp.float32).at[0, :P].set(probs)
    wlin_pad = jnp.pad(Wlin, ((0, 0), (0, 128 - PO)))
    blin_pad = jnp.zeros((8, 128), jnp.float32).at[0, :PO].set(blin)

    xp = jnp.pad(x.transpose(2, 0, 1), ((0, 0), (0, Npad - N), (0, 0)))

    msg = xp[:, src, :] * norm[None, :, None]
    axp = jnp.zeros((P, Npad, DIN), jnp.float32).at[:, dst, :].add(msg)

    out = _dense_stage(axp, xp, d2, wzh, bzh, probs_pad, wlin_pad, blin_pad)
    return out[:N, :PO]


# speedup vs baseline: 5.4561x; 2.5312x over previous
import functools

import jax
import jax.numpy as jnp
from jax import lax
from jax.experimental import pallas as pl
from jax.experimental.pallas import tpu as pltpu
from jax.experimental.pallas import tpu_sc as plsc

_TN = 512
_G = 128
_C = 512


def _make_agg(P, N, Npad, H, NCB, NC, NS):
    NW = NC * NS
    NPW = Npad // NW
    HV = H // 16
    ROWS = NPW * HV
    mesh = plsc.VectorSubcoreMesh(core_axis_name="c", subcore_axis_name="s")

    @functools.partial(
        pl.kernel,
        out_type=jax.ShapeDtypeStruct((P * Npad * HV, 16), jnp.float32),
        mesh=mesh,
        scratch_types=[
            pltpu.VMEM((48,), jnp.int32),
            pltpu.VMEM((2, _C + 16), jnp.int32),
            pltpu.VMEM((_C + 16,), jnp.float32),
            pltpu.VMEM((_G,), jnp.int32),
            pltpu.VMEM((_G, H), jnp.float32),
            pltpu.VMEM((ROWS, 16), jnp.float32),
            pltpu.SemaphoreType.DMA,
        ],
        compiler_params=pltpu.CompilerParams(
            needs_layout_passes=False, use_tc_tiling_on_sc=False),
    )
    def agg(xflat, eidx, enrm, offh, out, offv, ebi, ebn, idxv, rows, acc,
            sem):
        c = lax.axis_index("c")
        s = lax.axis_index("s")
        wid = s * NC + c
        base = wid * NPW
        pltpu.sync_copy(offh, offv)
        widv = jnp.zeros((16,), jnp.int32) + wid
        e0 = plsc.load_gather(offv, [widv])[0]
        e1 = plsc.load_gather(offv, [widv + 1])[0]
        cb0 = e0 // _C
        ncb = (e1 - cb0 * _C + (_C - 1)) // _C

        def period_body(t, carry):
            tb = t * N

            def zero_body(r, carry2):
                for u in range(8):
                    acc[r * 8 + u, :] = jnp.zeros((16,), jnp.float32)
                return carry2

            lax.fori_loop(0, ROWS // 8, zero_body, 0)

            def chunk_body(n, carry2):
                cb = cb0 + n
                pltpu.sync_copy(eidx.at[cb], ebi.at[:, pl.ds(0, _C)])
                pltpu.sync_copy(enrm.at[cb], ebn.at[pl.ds(0, _C)])
                ebase = cb * _C
                for g in range(_C // _G):
                    for j in range(_G // 16):
                        o = g * _G + j * 16
                        ev = lax.iota(jnp.int32, 16) + (ebase + o)
                        sv = ebi[0, pl.ds(o, 16)]
                        dv = ebi[1, pl.ds(o, 16)]
                        nv = ebn[pl.ds(o, 16)]
                        nv = jnp.where((ev >= e0) & (ev < e1), nv, 0.0)
                        lv = jnp.clip(dv - base, 0, NPW - 1)
                        idxv[pl.ds(j * 16, 16)] = sv + tb
                        ebn[pl.ds(o, 16)] = nv
                        ebi[1, pl.ds(o, 16)] = lv
                    pltpu.async_copy(xflat.at[idxv], rows, sem).wait()

                    def edge_body(i, carry3):
                        o = g * _G + i
                        oi = jnp.zeros((16,), jnp.int32) + o
                        wv = plsc.load_gather(ebn, [oi])
                        lrv = plsc.load_gather(
                            ebi, [jnp.ones((16,), jnp.int32), oi])
                        rb = lrv[0] * HV
                        for j in range(HV):
                            plsc.addupdate(
                                acc.at[rb + j, :],
                                wv * rows[i, pl.ds(j * 16, 16)])
                        return carry3

                    lax.fori_loop(0, _G, edge_body, 0)
                return carry2

            lax.fori_loop(0, ncb, chunk_body, 0)
            pltpu.sync_copy(acc, out.at[pl.ds((t * Npad + base) * HV, ROWS)])
            return carry

        lax.fori_loop(0, P, period_body, 0)

    return agg


def _dense_body(axp_ref, xp_ref, d2_ref, wzh_ref, bzh_ref, probs_ref,
                wlin_ref, blin_ref, out_ref):
    P = axp_ref.shape[0]
    H = wzh_ref.shape[0]
    d2 = d2_ref[...]
    wzh = wzh_ref[...]
    bzh = bzh_ref[0:1, :]
    acc = jnp.zeros((axp_ref.shape[1], H), jnp.float32)
    for t in range(P):
        y = axp_ref[t] + d2 * xp_ref[t]
        g = jnp.dot(y, wzh, preferred_element_type=jnp.float32) + bzh
        z = jax.nn.sigmoid(g[:, :H])
        ht = jnp.tanh(g[:, H:])
        acc = acc + probs_ref[0, t] * ((1.0 - z) * ht)
    h = jnp.maximum(acc, 0.0)
    out_ref[...] = (jnp.dot(h, wlin_ref[...], preferred_element_type=jnp.float32)
                    + blin_ref[0:1, :])


def _dense_stage(axp, xp, d2, wzh, bzh, probs_pad, wlin_pad, blin_pad):
    P, Npad, H = axp.shape
    grid = (Npad // _TN,)
    return pl.pallas_call(
        _dense_body,
        grid=grid,
        in_specs=[
            pl.BlockSpec((P, _TN, H), lambda i: (0, i, 0)),
            pl.BlockSpec((P, _TN, H), lambda i: (0, i, 0)),
            pl.BlockSpec((_TN, 1), lambda i: (i, 0)),
            pl.BlockSpec((H, 2 * H), lambda i: (0, 0)),
            pl.BlockSpec((8, 2 * H), lambda i: (0, 0)),
            pl.BlockSpec((8, 128), lambda i: (0, 0)),
            pl.BlockSpec((H, 128), lambda i: (0, 0)),
            pl.BlockSpec((8, 128), lambda i: (0, 0)),
        ],
        out_specs=pl.BlockSpec((_TN, 128), lambda i: (i, 0)),
        out_shape=jax.ShapeDtypeStruct((Npad, 128), jnp.float32),
        compiler_params=pltpu.CompilerParams(
            dimension_semantics=("parallel",)),
    )(axp, xp, d2, wzh, bzh, probs_pad, wlin_pad, blin_pad)


def kernel(x, edge_index, edge_attr, attention, Wz, bz, Lwz, lbz, Wr, br,
           Lwr, lbr, Wh, bh, Lwh, lbh, Wlin, blin):
    N, DIN, P = x.shape
    H = Wz.shape[1]
    PO = Wlin.shape[1]
    E = edge_index.shape[1]
    Npad = ((N + _TN - 1) // _TN) * _TN

    info = plsc.get_sparse_core_info()
    NC, NS = info.num_cores, info.num_subcores
    NW = NC * NS

    src = edge_index[0]
    dst = edge_index[1]
    w = edge_attr

    deg = jnp.zeros((N,), jnp.float32).at[dst].add(w) + 1.0
    dinv = jax.lax.rsqrt(deg)
    norm = dinv[src] * w * dinv[dst]
    d2 = (dinv * dinv)[:, None]
    d2 = jnp.pad(d2, ((0, Npad - N), (0, 0)))

    wzh = jnp.concatenate([Wz @ Lwz[:H], Wh @ Lwh[:H]], axis=1)
    bzh = jnp.concatenate([bz @ Lwz[:H] + lbz, bh @ Lwh[:H] + lbh])
    bzh = jnp.zeros((8, 2 * H), jnp.float32).at[0].set(bzh)
    probs = jax.nn.softmax(attention)
    probs_pad = jnp.zeros((8, 128), jnp.float32).at[0, :P].set(probs)
    wlin_pad = jnp.pad(Wlin, ((0, 0), (0, 128 - PO)))
    blin_pad = jnp.zeros((8, 128), jnp.float32).at[0, :PO].set(blin)

    xpnp = x.transpose(2, 0, 1)
    xp = jnp.pad(xpnp, ((0, 0), (0, Npad - N), (0, 0)))

    order = jnp.argsort(dst)
    srcs = src[order]
    dsts = dst[order]
    nrms = norm[order]
    NCB = (E + _C - 1) // _C
    Ep = NCB * _C
    srcp = jnp.zeros((Ep,), jnp.int32).at[:E].set(srcs)
    dstp = jnp.zeros((Ep,), jnp.int32).at[:E].set(dsts)
    nrmp = jnp.zeros((Ep,), jnp.float32).at[:E].set(nrms)
    eidx = jnp.stack([srcp.reshape(NCB, _C), dstp.reshape(NCB, _C)], axis=1)
    enrm = nrmp.reshape(NCB, _C)
    NPW = Npad // NW
    bounds = jnp.arange(NW + 1, dtype=jnp.int32) * NPW
    off = jnp.searchsorted(dsts, bounds).astype(jnp.int32)
    off48 = jnp.zeros((48,), jnp.int32).at[:NW + 1].set(off)

    xflat = xpnp.reshape(P * N, H)
    aggflat = _make_agg(P, N, Npad, H, NCB, NC, NS)(xflat, eidx, enrm, off48)
    axp = aggflat.reshape(P, Npad, H)

    out = _dense_stage(axp, xp, d2, wzh, bzh, probs_pad, wlin_pad, blin_pad)
    return out[:N, :PO]


# speedup vs baseline: 6.2528x; 1.1460x over previous
import functools

import jax
import jax.numpy as jnp
from jax import lax
from jax.experimental import pallas as pl
from jax.experimental.pallas import tpu as pltpu
from jax.experimental.pallas import tpu_sc as plsc

_TN = 512
_G = 64
_C = 512


def _make_agg(P, N, Npad, H, NCB, NC, NS):
    NW = NC * NS
    NPW = Npad // NW
    HV = H // 16
    ROWS = NPW * HV
    mesh = plsc.VectorSubcoreMesh(core_axis_name="c", subcore_axis_name="s")

    @functools.partial(
        pl.kernel,
        out_type=jax.ShapeDtypeStruct((P * Npad * HV, 16), jnp.float32),
        mesh=mesh,
        scratch_types=[
            pltpu.VMEM((48,), jnp.int32),
            pltpu.VMEM((2, _C), jnp.int32),
            pltpu.VMEM((_C,), jnp.float32),
            pltpu.VMEM((_C,), jnp.int32),
            pltpu.VMEM((_G, H), jnp.float32),
            pltpu.VMEM((_G, H), jnp.float32),
            pltpu.VMEM((ROWS, 16), jnp.float32),
            pltpu.SemaphoreType.DMA,
            pltpu.SemaphoreType.DMA,
        ],
        compiler_params=pltpu.CompilerParams(
            needs_layout_passes=False, use_tc_tiling_on_sc=False),
    )
    def agg(xflat, eidx, enrm, offh, out, offv, ebi, ebn, idxs, rows_a,
            rows_b, acc, sem_a, sem_b):
        c = lax.axis_index("c")
        s = lax.axis_index("s")
        wid = s * NC + c
        base = wid * NPW
        pltpu.sync_copy(offh, offv)
        widv = jnp.zeros((16,), jnp.int32) + wid
        e0 = plsc.load_gather(offv, [widv])[0]
        e1 = plsc.load_gather(offv, [widv + 1])[0]
        cb0 = e0 // _C
        ncb = (e1 - cb0 * _C + (_C - 1)) // _C
        NG = _C // _G
        bufs = (rows_a, rows_b)
        sems = (sem_a, sem_b)

        def period_body(t, carry):
            tb = t * N

            def zero_body(r, carry2):
                for u in range(8):
                    acc[r * 8 + u, :] = jnp.zeros((16,), jnp.float32)
                return carry2

            lax.fori_loop(0, ROWS // 8, zero_body, 0)

            def chunk_body(n, carry2):
                cb = cb0 + n
                pltpu.sync_copy(eidx.at[cb], ebi)
                pltpu.sync_copy(enrm.at[cb], ebn)
                ebase = cb * _C
                for j in range(_C // 16):
                    o = j * 16
                    ev = lax.iota(jnp.int32, 16) + (ebase + o)
                    sv = ebi[0, pl.ds(o, 16)]
                    dv = ebi[1, pl.ds(o, 16)]
                    nv = ebn[pl.ds(o, 16)]
                    nv = jnp.where((ev >= e0) & (ev < e1), nv, 0.0)
                    lv = jnp.clip(dv - base, 0, NPW - 1)
                    idxs[pl.ds(o, 16)] = sv + tb
                    ebn[pl.ds(o, 16)] = nv
                    ebi[1, pl.ds(o, 16)] = lv
                cps = [None, None]
                cps[0] = pltpu.async_copy(
                    xflat.at[idxs.at[pl.ds(0, _G)]], rows_a, sem_a)
                for g in range(NG):
                    if g + 1 < NG:
                        cps[(g + 1) % 2] = pltpu.async_copy(
                            xflat.at[idxs.at[pl.ds((g + 1) * _G, _G)]],
                            bufs[(g + 1) % 2], sems[(g + 1) % 2])
                    cps[g % 2].wait()
                    buf = bufs[g % 2]

                    def acc_sub(k, carry3, g=g, buf=buf):
                        o16 = pl.multiple_of(g * _G + k * 16, 16)
                        nv = ebn[pl.ds(o16, 16)]
                        lv = ebi[1, pl.ds(o16, 16)]
                        for u in range(16):
                            w_ = nv[u]
                            rb = lv[u] * HV
                            r = k * 16 + u
                            for j in range(HV):
                                plsc.addupdate(
                                    acc.at[rb + j, :],
                                    w_ * buf[r, pl.ds(j * 16, 16)])
                        return carry3

                    lax.fori_loop(0, _G // 16, acc_sub, 0)
                return carry2

            lax.fori_loop(0, ncb, chunk_body, 0)
            pltpu.sync_copy(acc, out.at[pl.ds((t * Npad + base) * HV, ROWS)])
            return carry

        lax.fori_loop(0, P, period_body, 0)

    return agg


def _dense_body(axp_ref, xp_ref, d2_ref, wzh_ref, bzh_ref, probs_ref,
                wlin_ref, blin_ref, out_ref):
    P = axp_ref.shape[0]
    H = wzh_ref.shape[0]
    d2 = d2_ref[...]
    wzh = wzh_ref[...]
    bzh = bzh_ref[0:1, :]
    acc = jnp.zeros((axp_ref.shape[1], H), jnp.float32)
    for t in range(P):
        y = axp_ref[t] + d2 * xp_ref[t]
        g = jnp.dot(y, wzh, preferred_element_type=jnp.float32) + bzh
        z = jax.nn.sigmoid(g[:, :H])
        ht = jnp.tanh(g[:, H:])
        acc = acc + probs_ref[0, t] * ((1.0 - z) * ht)
    h = jnp.maximum(acc, 0.0)
    out_ref[...] = (jnp.dot(h, wlin_ref[...], preferred_element_type=jnp.float32)
                    + blin_ref[0:1, :])


def _dense_stage(axp, xp, d2, wzh, bzh, probs_pad, wlin_pad, blin_pad):
    P, Npad, H = axp.shape
    grid = (Npad // _TN,)
    return pl.pallas_call(
        _dense_body,
        grid=grid,
        in_specs=[
            pl.BlockSpec((P, _TN, H), lambda i: (0, i, 0)),
            pl.BlockSpec((P, _TN, H), lambda i: (0, i, 0)),
            pl.BlockSpec((_TN, 1), lambda i: (i, 0)),
            pl.BlockSpec((H, 2 * H), lambda i: (0, 0)),
            pl.BlockSpec((8, 2 * H), lambda i: (0, 0)),
            pl.BlockSpec((8, 128), lambda i: (0, 0)),
            pl.BlockSpec((H, 128), lambda i: (0, 0)),
            pl.BlockSpec((8, 128), lambda i: (0, 0)),
        ],
        out_specs=pl.BlockSpec((_TN, 128), lambda i: (i, 0)),
        out_shape=jax.ShapeDtypeStruct((Npad, 128), jnp.float32),
        compiler_params=pltpu.CompilerParams(
            dimension_semantics=("parallel",)),
    )(axp, xp, d2, wzh, bzh, probs_pad, wlin_pad, blin_pad)


def kernel(x, edge_index, edge_attr, attention, Wz, bz, Lwz, lbz, Wr, br,
           Lwr, lbr, Wh, bh, Lwh, lbh, Wlin, blin):
    N, DIN, P = x.shape
    H = Wz.shape[1]
    PO = Wlin.shape[1]
    E = edge_index.shape[1]
    Npad = ((N + _TN - 1) // _TN) * _TN

    info = plsc.get_sparse_core_info()
    NC, NS = info.num_cores, info.num_subcores
    NW = NC * NS

    src = edge_index[0]
    dst = edge_index[1]
    w = edge_attr

    deg = jnp.zeros((N,), jnp.float32).at[dst].add(w) + 1.0
    dinv = jax.lax.rsqrt(deg)
    norm = dinv[src] * w * dinv[dst]
    d2 = (dinv * dinv)[:, None]
    d2 = jnp.pad(d2, ((0, Npad - N), (0, 0)))

    wzh = jnp.concatenate([Wz @ Lwz[:H], Wh @ Lwh[:H]], axis=1)
    bzh = jnp.concatenate([bz @ Lwz[:H] + lbz, bh @ Lwh[:H] + lbh])
    bzh = jnp.zeros((8, 2 * H), jnp.float32).at[0].set(bzh)
    probs = jax.nn.softmax(attention)
    probs_pad = jnp.zeros((8, 128), jnp.float32).at[0, :P].set(probs)
    wlin_pad = jnp.pad(Wlin, ((0, 0), (0, 128 - PO)))
    blin_pad = jnp.zeros((8, 128), jnp.float32).at[0, :PO].set(blin)

    xpnp = x.transpose(2, 0, 1)
    xp = jnp.pad(xpnp, ((0, 0), (0, Npad - N), (0, 0)))

    order = jnp.argsort(dst)
    srcs = src[order]
    dsts = dst[order]
    nrms = norm[order]
    NCB = (E + _C - 1) // _C
    Ep = NCB * _C
    srcp = jnp.zeros((Ep,), jnp.int32).at[:E].set(srcs)
    dstp = jnp.zeros((Ep,), jnp.int32).at[:E].set(dsts)
    nrmp = jnp.zeros((Ep,), jnp.float32).at[:E].set(nrms)
    eidx = jnp.stack([srcp.reshape(NCB, _C), dstp.reshape(NCB, _C)], axis=1)
    enrm = nrmp.reshape(NCB, _C)
    NPW = Npad // NW
    bounds = jnp.arange(NW + 1, dtype=jnp.int32) * NPW
    off = jnp.searchsorted(dsts, bounds).astype(jnp.int32)
    off48 = jnp.zeros((48,), jnp.int32).at[:NW + 1].set(off)

    xflat = xpnp.reshape(P * N, H)
    aggflat = _make_agg(P, N, Npad, H, NCB, NC, NS)(xflat, eidx, enrm, off48)
    axp = aggflat.reshape(P, Npad, H)

    out = _dense_stage(axp, xp, d2, wzh, bzh, probs_pad, wlin_pad, blin_pad)
    return out[:N, :PO]
